# baseline (speedup 1.0000x reference)
import functools

import jax
import jax.numpy as jnp
from jax import lax
from jax.experimental import pallas as pl
from jax.experimental.pallas import tpu as pltpu
from jax.experimental.pallas import tpu_sc as plsc

NUM_CORES = 2
NUM_SUBCORES = 16
NUM_WORKERS = NUM_CORES * NUM_SUBCORES
GRP = 4
CPG = 2


@functools.lru_cache(maxsize=None)
def _build(batch: int, seq: int, d: int):
    rows_per_w = batch // NUM_WORKERS
    ngroup = rows_per_w // (GRP * CPG)
    assert ngroup * GRP * CPG == rows_per_w and ngroup >= 4 and ngroup % 2 == 0

    @functools.partial(
        pl.kernel,
        mesh=plsc.VectorSubcoreMesh(core_axis_name="c", subcore_axis_name="s"),
        out_type=jax.ShapeDtypeStruct((batch, seq, d), jnp.float32),
        scratch_types=[
            pltpu.VMEM((rows_per_w * seq,), jnp.int32),
        ]
        + [pltpu.VMEM((GRP * seq, d), jnp.float32)] * (2 * CPG)
        + [pltpu.SemaphoreType.DMA] * (4 * CPG),
        compiler_params=pltpu.CompilerParams(use_tc_tiling_on_sc=True),
    )
    def emb_kernel(xf_hbm, table_hbm, out_hbm, idx_v, *scratch):
        bufs = scratch[: 2 * CPG]
        gsems = scratch[2 * CPG : 4 * CPG]
        osems = scratch[4 * CPG :]
        pair0 = tuple(range(CPG))
        pair1 = tuple(range(CPG, 2 * CPG))

        wid = lax.axis_index("s") * NUM_CORES + lax.axis_index("c")
        base = wid * rows_per_w
        pltpu.sync_copy(xf_hbm.at[pl.ds(base * seq, rows_per_w * seq)], idx_v)

        def fire_gather(row, b):
            pltpu.async_copy(
                table_hbm.at[idx_v.at[pl.ds(row * seq, GRP * seq)]], bufs[b], gsems[b]
            )

        def wait_gather(b):
            pltpu.make_async_copy(
                table_hbm.at[idx_v.at[pl.ds(0, GRP * seq)]], bufs[b], gsems[b]
            ).wait()

        def fire_out(row, b):
            pltpu.async_copy(
                bufs[b].reshape(GRP, seq, d),
                out_hbm.at[pl.ds(base + row, GRP)],
                osems[b],
            )

        def wait_out(b):
            pltpu.make_async_copy(
                bufs[b].reshape(GRP, seq, d),
                out_hbm.at[pl.ds(base, GRP)],
                osems[b],
            ).wait()

        def phase(g, cur, nxt, first=False, last=False):
            for b in cur:
                wait_gather(b)
            if not first:
                for b in nxt:
                    wait_out(b)
            if not last:
                for i, b in enumerate(nxt):
                    fire_gather(((g + 1) * CPG + i) * GRP, b)
            for i, b in enumerate(cur):
                fire_out((g * CPG + i) * GRP, b)

        for i, b in enumerate(pair0):
            fire_gather(i * GRP, b)
        phase(0, pair0, pair1, first=True)

        @pl.loop(1, ngroup - 1, step=2)
        def _(g):
            phase(g, pair1, pair0)
            phase(g + 1, pair0, pair1)

        phase(ngroup - 1, pair1, pair0, last=True)
        for b in pair1:
            wait_out(b)

    return emb_kernel


def kernel(x, table):
    batch, seq = x.shape
    d = table.shape[1]
    assert batch % NUM_WORKERS == 0
    xf = x.astype(jnp.int32).reshape(batch * seq)
    return _build(batch, seq, d)(xf, table)


# speedup vs baseline: 1.7958x; 1.7958x over previous
import functools

import jax
import jax.numpy as jnp
from jax import lax
from jax.experimental import pallas as pl
from jax.experimental.pallas import tpu as pltpu
from jax.experimental.pallas import tpu_sc as plsc

NUM_CORES = 2
NUM_SUBCORES = 16
NUM_WORKERS = NUM_CORES * NUM_SUBCORES
CHUNK = 128
CPG = 2


@functools.lru_cache(maxsize=None)
def _build(batch: int, seq: int, d: int):
    total = batch * seq
    colw = batch // NUM_WORKERS
    split = colw // CHUNK
    nchunk = seq * split
    ngroup = nchunk // CPG
    assert split * CHUNK == colw and ngroup * CPG == nchunk and ngroup >= 4

    @functools.partial(
        pl.kernel,
        mesh=plsc.VectorSubcoreMesh(core_axis_name="c", subcore_axis_name="s"),
        out_type=jax.ShapeDtypeStruct((total, d), jnp.float32),
        scratch_types=[
            pltpu.VMEM((seq, colw), jnp.int32),
        ]
        + [pltpu.VMEM((CHUNK, d), jnp.float32)] * (2 * CPG)
        + [pltpu.SemaphoreType.DMA] * (4 * CPG),
    )
    def emb_kernel(xt_hbm, table_hbm, out_hbm, idx_v, *scratch):
        bufs = scratch[: 2 * CPG]
        gsems = scratch[2 * CPG : 4 * CPG]
        osems = scratch[4 * CPG :]
        pair0 = tuple(range(CPG))
        pair1 = tuple(range(CPG, 2 * CPG))

        wid = lax.axis_index("s") * NUM_CORES + lax.axis_index("c")
        col0 = wid * colw
        pltpu.sync_copy(xt_hbm.at[:, pl.ds(col0, colw)], idx_v)

        def fire_gather(c, b):
            s, off = c // split, (c % split) * CHUNK
            pltpu.async_copy(
                table_hbm.at[idx_v.at[s, pl.ds(off, CHUNK)]], bufs[b], gsems[b]
            )

        def wait_gather(b):
            pltpu.make_async_copy(
                table_hbm.at[idx_v.at[0, pl.ds(0, CHUNK)]], bufs[b], gsems[b]
            ).wait()

        def fire_out(c, b):
            s, off = c // split, (c % split) * CHUNK
            pltpu.async_copy(
                bufs[b],
                out_hbm.at[pl.ds(s * batch + col0 + off, CHUNK)],
                osems[b],
            )

        def wait_out(b):
            pltpu.make_async_copy(
                bufs[b], out_hbm.at[pl.ds(col0, CHUNK)], osems[b]
            ).wait()

        def phase(g, cur, nxt, first=False, last=False):
            for b in cur:
                wait_gather(b)
            if not first:
                for b in nxt:
                    wait_out(b)
            if not last:
                for i, b in enumerate(nxt):
                    fire_gather((g + 1) * CPG + i, b)
            for i, b in enumerate(cur):
                fire_out(g * CPG + i, b)

        for i, b in enumerate(pair0):
            fire_gather(i, b)
        phase(0, pair0, pair1, first=True)

        if ngroup % 2 == 0:
            @pl.loop(1, ngroup - 1, step=2)
            def _(g):
                phase(g, pair1, pair0)
                phase(g + 1, pair0, pair1)

            phase(ngroup - 1, pair1, pair0, last=True)
            for b in pair1:
                wait_out(b)
        else:
            @pl.loop(1, ngroup - 2, step=2)
            def _(g):
                phase(g, pair1, pair0)
                phase(g + 1, pair0, pair1)

            phase(ngroup - 2, pair1, pair0)
            phase(ngroup - 1, pair0, pair1, last=True)
            for b in pair0:
                wait_out(b)

    return emb_kernel


def kernel(x, table):
    batch, seq = x.shape
    d = table.shape[1]
    xt = x.astype(jnp.int32).T
    out = _build(batch, seq, d)(xt, table)
    return out.reshape(seq, batch, d).transpose(1, 0, 2)
